# baseline (device time: 72582 ns/iter reference)
import jax
import jax.numpy as jnp
from jax import lax
from jax.experimental import pallas as pl
from jax.experimental.pallas import tpu as pltpu

N_DEV = 8
B = 2048
D = 256
BP = B // N_DEV
HB = BP // 2
N_SEM = 14 * 7

ARRIVAL = (1, 3, 4, 2, 5, 7, 6)
ISSUE = tuple(reversed(ARRIVAL))


def kernel(x, Win0, Wout0, Win1, Wout1, Win2, Wout2):
    def body(x_ref, win0_ref, wout0_ref, win1_ref, wout1_ref, win2_ref,
             wout2_ref, out_ref, xfull, xns, sbufs, rbufs,
             send_sems, recv_sems):
        p = lax.axis_index("i").astype(jnp.int32)

        barrier = pltpu.get_barrier_semaphore()
        for d in range(1, N_DEV):
            pl.semaphore_signal(barrier, inc=1, device_id=(p ^ d,),
                                device_id_type=pl.DeviceIdType.MESH)
        pl.semaphore_wait(barrier, N_DEV - 1)

        all_rdmas = []

        def half_sl(j, h):
            return pl.ds(j * BP + h * HB, HB)

        def bcast_half(buf, h, base):
            rdmas = {}
            for d in ISSUE:
                r = pltpu.make_async_remote_copy(
                    src_ref=buf.at[half_sl(p, h)], dst_ref=buf.at[half_sl(p, h)],
                    send_sem=send_sems.at[base + 2 * (d - 1) + h],
                    recv_sem=recv_sems.at[base + 2 * (d - 1) + h],
                    device_id=(p ^ d,), device_id_type=pl.DeviceIdType.MESH,
                )
                r.start()
                rdmas[(d, h)] = r
                all_rdmas.append(r)
            return rdmas

        my_sl = pl.ds(p * BP, BP)
        xfull[my_sl, :] = x_ref[:, :].astype(jnp.bfloat16)
        prev_ag = {}
        for h in (0, 1):
            prev_ag.update(bcast_half(xfull, h, 0))
        prev_buf = xfull

        wins = [win0_ref, win1_ref, win2_ref]
        wouts = [wout0_ref, wout1_ref, wout2_ref]
        own_in = x_ref[:, :].astype(jnp.bfloat16)
        red_halves = [None, None]
        for l in range(3):
            w_in = wins[l][:, :].astype(jnp.bfloat16)
            w_out = wouts[l][:, :].astype(jnp.bfloat16)
            sbuf = sbufs.at[l]
            rs_base = 14 + 28 * l
            ag_base = rs_base + 14

            def block_partial(in_j):
                hj = jnp.dot(in_j, w_in, preferred_element_type=jnp.float32)
                hj = jnp.maximum(hj, 0.0).astype(jnp.bfloat16)
                return jnp.dot(hj, w_out, preferred_element_type=jnp.float32)

            own_acc = block_partial(own_in)

            rs_rdmas = {}
            for d in ARRIVAL:
                j = p ^ d
                for h in (0, 1):
                    prev_ag[(d, h)].wait_recv()
                    acc_jh = block_partial(prev_buf[half_sl(j, h), :])
                    sbuf[half_sl(j, h), :] = acc_jh.astype(jnp.bfloat16)
                    r = pltpu.make_async_remote_copy(
                        src_ref=sbuf.at[half_sl(j, h)],
                        dst_ref=rbufs.at[l, d, pl.ds(h * HB, HB)],
                        send_sem=send_sems.at[rs_base + 2 * (d - 1) + h],
                        recv_sem=recv_sems.at[rs_base + 2 * (d - 1) + h],
                        device_id=(j,), device_id_type=pl.DeviceIdType.MESH,
                    )
                    r.start()
                    rs_rdmas[(d, h)] = r
                    all_rdmas.append(r)

            xn = xns.at[l]
            prev_ag = {}
            for h in (0, 1):
                redh = own_acc[h * HB:(h + 1) * HB, :]
                for d in ARRIVAL:
                    rs_rdmas[(d, h)].wait_recv()
                    redh = redh + rbufs[l, d, pl.ds(h * HB, HB), :].astype(jnp.float32)
                red_halves[h] = redh
                xn[half_sl(p, h), :] = redh.astype(jnp.bfloat16)
                prev_ag.update(bcast_half(xn, h, ag_base))
            own_in = xn[my_sl, :]
            prev_buf = xn

        for h in (0, 1):
            out_ref[half_sl(p, h), :] = red_halves[h]
        for d in ARRIVAL:
            for h in (0, 1):
                prev_ag[(d, h)].wait_recv()
                sl = half_sl(p ^ d, h)
                out_ref[sl, :] = prev_buf[sl, :].astype(jnp.float32)

        for r in all_rdmas:
            r.wait_send()

    return pl.pallas_call(
        body,
        out_shape=jax.ShapeDtypeStruct((B, D), jnp.float32),
        in_specs=[pl.BlockSpec(memory_space=pltpu.VMEM)] * 7,
        out_specs=pl.BlockSpec(memory_space=pltpu.VMEM),
        scratch_shapes=[
            pltpu.VMEM((B, D), jnp.bfloat16),
            pltpu.VMEM((3, B, D), jnp.bfloat16),
            pltpu.VMEM((3, B, D), jnp.bfloat16),
            pltpu.VMEM((3, N_DEV, BP, D), jnp.bfloat16),
            pltpu.SemaphoreType.DMA((N_SEM,)),
            pltpu.SemaphoreType.DMA((N_SEM,)),
        ],
        compiler_params=pltpu.CompilerParams(collective_id=0),
    )(x, Win0, Wout0, Win1, Wout1, Win2, Wout2)


# device time: 66743 ns/iter; 1.0875x vs baseline; 1.0875x over previous
import jax
import jax.numpy as jnp
from jax import lax
from jax.experimental import pallas as pl
from jax.experimental.pallas import tpu as pltpu

N_DEV = 8
B = 2048
D = 256
BP = B // N_DEV
HB = BP // 2
N_SEM = 14 * 7

ARRIVAL = (1, 3, 4, 2, 5, 7, 6)
ISSUE = tuple(range(1, N_DEV))


def kernel(x, Win0, Wout0, Win1, Wout1, Win2, Wout2):
    def body(x_ref, win0_ref, wout0_ref, win1_ref, wout1_ref, win2_ref,
             wout2_ref, out_ref, xfull, xns, sbufs, rbufs,
             send_sems, recv_sems):
        p = lax.axis_index("i").astype(jnp.int32)

        barrier = pltpu.get_barrier_semaphore()
        for d in range(1, N_DEV):
            pl.semaphore_signal(barrier, inc=1, device_id=(p ^ d,),
                                device_id_type=pl.DeviceIdType.MESH)
        pl.semaphore_wait(barrier, N_DEV - 1)

        all_rdmas = []

        def half_sl(j, h):
            return pl.ds(j * BP + h * HB, HB)

        def bcast_half(buf, h, base):
            rdmas = {}
            for d in ISSUE:
                r = pltpu.make_async_remote_copy(
                    src_ref=buf.at[half_sl(p, h)], dst_ref=buf.at[half_sl(p, h)],
                    send_sem=send_sems.at[base + 2 * (d - 1) + h],
                    recv_sem=recv_sems.at[base + 2 * (d - 1) + h],
                    device_id=(p ^ d,), device_id_type=pl.DeviceIdType.MESH,
                )
                r.start()
                rdmas[(d, h)] = r
                all_rdmas.append(r)
            return rdmas

        my_sl = pl.ds(p * BP, BP)
        xfull[my_sl, :] = x_ref[:, :].astype(jnp.bfloat16)
        prev_ag = {}
        for h in (0, 1):
            prev_ag.update(bcast_half(xfull, h, 0))
        prev_buf = xfull

        wins = [win0_ref, win1_ref, win2_ref]
        wouts = [wout0_ref, wout1_ref, wout2_ref]
        own_in = x_ref[:, :].astype(jnp.bfloat16)
        red_halves = [None, None]
        for l in range(3):
            w_in = wins[l][:, :].astype(jnp.bfloat16)
            w_out = wouts[l][:, :].astype(jnp.bfloat16)
            sbuf = sbufs.at[l]
            rs_base = 14 + 28 * l
            ag_base = rs_base + 14

            def block_partial(in_j):
                hj = jnp.dot(in_j, w_in, preferred_element_type=jnp.float32)
                hj = jnp.maximum(hj, 0.0).astype(jnp.bfloat16)
                return jnp.dot(hj, w_out, preferred_element_type=jnp.float32)

            own_acc = block_partial(own_in)

            rs_rdmas = {}
            for d in ARRIVAL:
                j = p ^ d
                for h in (0, 1):
                    prev_ag[(d, h)].wait_recv()
                    acc_jh = block_partial(prev_buf[half_sl(j, h), :])
                    sbuf[half_sl(j, h), :] = acc_jh.astype(jnp.bfloat16)
                    r = pltpu.make_async_remote_copy(
                        src_ref=sbuf.at[half_sl(j, h)],
                        dst_ref=rbufs.at[l, d, pl.ds(h * HB, HB)],
                        send_sem=send_sems.at[rs_base + 2 * (d - 1) + h],
                        recv_sem=recv_sems.at[rs_base + 2 * (d - 1) + h],
                        device_id=(j,), device_id_type=pl.DeviceIdType.MESH,
                    )
                    r.start()
                    rs_rdmas[(d, h)] = r
                    all_rdmas.append(r)

            xn = xns.at[l]
            prev_ag = {}
            for h in (0, 1):
                redh = own_acc[h * HB:(h + 1) * HB, :]
                for d in ARRIVAL:
                    rs_rdmas[(d, h)].wait_recv()
                    redh = redh + rbufs[l, d, pl.ds(h * HB, HB), :].astype(jnp.float32)
                red_halves[h] = redh
                xn[half_sl(p, h), :] = redh.astype(jnp.bfloat16)
                prev_ag.update(bcast_half(xn, h, ag_base))
            own_in = xn[my_sl, :]
            prev_buf = xn

        for h in (0, 1):
            out_ref[half_sl(p, h), :] = red_halves[h]
        for d in ARRIVAL:
            for h in (0, 1):
                prev_ag[(d, h)].wait_recv()
                sl = half_sl(p ^ d, h)
                out_ref[sl, :] = prev_buf[sl, :].astype(jnp.float32)

        for r in all_rdmas:
            r.wait_send()

    return pl.pallas_call(
        body,
        out_shape=jax.ShapeDtypeStruct((B, D), jnp.float32),
        in_specs=[pl.BlockSpec(memory_space=pltpu.VMEM)] * 7,
        out_specs=pl.BlockSpec(memory_space=pltpu.VMEM),
        scratch_shapes=[
            pltpu.VMEM((B, D), jnp.bfloat16),
            pltpu.VMEM((3, B, D), jnp.bfloat16),
            pltpu.VMEM((3, B, D), jnp.bfloat16),
            pltpu.VMEM((3, N_DEV, BP, D), jnp.bfloat16),
            pltpu.SemaphoreType.DMA((N_SEM,)),
            pltpu.SemaphoreType.DMA((N_SEM,)),
        ],
        compiler_params=pltpu.CompilerParams(collective_id=0),
    )(x, Win0, Wout0, Win1, Wout1, Win2, Wout2)
